# baseline (speedup 1.0000x reference)
import functools

import jax
import jax.numpy as jnp
from jax.experimental import pallas as pl
from jax.experimental.pallas import tpu as pltpu


def _fused_body(x_ref, w_ref, adj_ref, out_ref, sup_ref, *, nk, tk):
    m = pl.program_id(1)
    k = pl.program_id(2)

    @pl.when(m == 0)
    def _():
        sup_ref[pl.ds(k * tk, tk), :] = jnp.dot(
            x_ref[pl.ds(k * tk, tk), :], w_ref[...],
            preferred_element_type=jnp.float32,
        ).astype(jnp.bfloat16)

    adj_blk = adj_ref[...].astype(jnp.bfloat16)
    sup_blk = sup_ref[pl.ds(k * tk, tk), :]
    prod = jnp.dot(adj_blk, sup_blk, preferred_element_type=jnp.float32)

    @pl.when(k == 0)
    def _():
        out_ref[...] = prod

    @pl.when(k > 0)
    def _():
        out_ref[...] += prod

    @pl.when(k == nk - 1)
    def _():
        out_ref[...] = jnp.maximum(out_ref[...], 0.0)


@jax.jit
def kernel(input, adj, W):
    n_nodes, in_features = input.shape
    out_features = W.shape[1]

    tm, tk = 512, 2048
    ncore = 2
    nm = n_nodes // tm // ncore
    nk = n_nodes // tk

    out = pl.pallas_call(
        functools.partial(_fused_body, nk=nk, tk=tk),
        grid=(ncore, nm, nk),
        in_specs=[
            pl.BlockSpec((n_nodes, in_features), lambda c, m, k: (0, 0)),
            pl.BlockSpec((in_features, out_features), lambda c, m, k: (0, 0)),
            pl.BlockSpec((tm, tk), lambda c, m, k, nm=nm: (c * nm + m, k)),
        ],
        out_specs=pl.BlockSpec(
            (tm, out_features), lambda c, m, k, nm=nm: (c * nm + m, 0)),
        out_shape=jax.ShapeDtypeStruct((n_nodes, out_features), jnp.float32),
        scratch_shapes=[pltpu.VMEM((n_nodes, out_features), jnp.bfloat16)],
        compiler_params=pltpu.CompilerParams(
            dimension_semantics=("parallel", "arbitrary", "arbitrary")),
    )(input, W, adj)
    return out


# speedup vs baseline: 1.2266x; 1.2266x over previous
import functools

import jax
import jax.numpy as jnp
from jax.experimental import pallas as pl
from jax.experimental.pallas import tpu as pltpu


def _fused_body(x_ref, w_ref, adj_ref, out_ref, sup_ref, *, nk, tk):
    m = pl.program_id(1)
    k = pl.program_id(2)

    @pl.when((m == 0) & (k == 0))
    def _():
        sup_ref[...] = jnp.dot(
            x_ref[...], w_ref[...], preferred_element_type=jnp.float32
        ).astype(jnp.bfloat16)

    @pl.when(k == 0)
    def _():
        out_ref[...] = jnp.zeros_like(out_ref)

    adj_blk = adj_ref[...].astype(jnp.bfloat16)
    sup_blk = sup_ref[pl.ds(k * tk, tk), :]
    out_ref[...] += jnp.dot(adj_blk, sup_blk,
                            preferred_element_type=jnp.float32)

    @pl.when(k == nk - 1)
    def _():
        out_ref[...] = jnp.maximum(out_ref[...], 0.0)


@jax.jit
def kernel(input, adj, W):
    n_nodes, in_features = input.shape
    out_features = W.shape[1]

    tm, tk = 1024, 2048
    ncore = 2
    nm = n_nodes // tm // ncore
    nk = n_nodes // tk

    out = pl.pallas_call(
        functools.partial(_fused_body, nk=nk, tk=tk),
        grid=(ncore, nm, nk),
        in_specs=[
            pl.BlockSpec((n_nodes, in_features), lambda c, m, k: (0, 0)),
            pl.BlockSpec((in_features, out_features), lambda c, m, k: (0, 0)),
            pl.BlockSpec((tm, tk), lambda c, m, k, nm=nm: (c * nm + m, k)),
        ],
        out_specs=pl.BlockSpec(
            (tm, out_features), lambda c, m, k, nm=nm: (c * nm + m, 0)),
        out_shape=jax.ShapeDtypeStruct((n_nodes, out_features), jnp.float32),
        scratch_shapes=[pltpu.VMEM((n_nodes, out_features), jnp.bfloat16)],
        compiler_params=pltpu.CompilerParams(
            dimension_semantics=("parallel", "arbitrary", "arbitrary")),
    )(input, W, adj)
    return out


# speedup vs baseline: 1.2284x; 1.0014x over previous
import functools

import jax
import jax.numpy as jnp
from jax.experimental import pallas as pl
from jax.experimental.pallas import tpu as pltpu


def _fused_body(x_ref, w_ref, adj_ref, out_ref, sup_ref, *, nk, tk):
    m = pl.program_id(1)
    k = pl.program_id(2)

    @pl.when((m == 0) & (k == 0))
    def _():
        sup_ref[...] = jnp.dot(
            x_ref[...], w_ref[...], preferred_element_type=jnp.float32
        ).astype(jnp.bfloat16)

    @pl.when(k == 0)
    def _():
        out_ref[...] = jnp.zeros_like(out_ref)

    adj_blk = adj_ref[...].astype(jnp.bfloat16)
    sup_blk = sup_ref[pl.ds(k * tk, tk), :]
    out_ref[...] += jnp.dot(adj_blk, sup_blk,
                            preferred_element_type=jnp.float32)

    @pl.when(k == nk - 1)
    def _():
        out_ref[...] = jnp.maximum(out_ref[...], 0.0)


@jax.jit
def kernel(input, adj, W):
    n_nodes, in_features = input.shape
    out_features = W.shape[1]

    tm, tk = 2048, 2048
    ncore = 2
    nm = n_nodes // tm // ncore
    nk = n_nodes // tk

    out = pl.pallas_call(
        functools.partial(_fused_body, nk=nk, tk=tk),
        grid=(ncore, nm, nk),
        in_specs=[
            pl.BlockSpec((n_nodes, in_features), lambda c, m, k: (0, 0)),
            pl.BlockSpec((in_features, out_features), lambda c, m, k: (0, 0)),
            pl.BlockSpec((tm, tk), lambda c, m, k, nm=nm: (c * nm + m, k)),
        ],
        out_specs=pl.BlockSpec(
            (tm, out_features), lambda c, m, k, nm=nm: (c * nm + m, 0)),
        out_shape=jax.ShapeDtypeStruct((n_nodes, out_features), jnp.float32),
        scratch_shapes=[pltpu.VMEM((n_nodes, out_features), jnp.bfloat16)],
        compiler_params=pltpu.CompilerParams(
            dimension_semantics=("parallel", "arbitrary", "arbitrary")),
    )(input, W, adj)
    return out


# speedup vs baseline: 1.2375x; 1.0074x over previous
import functools

import jax
import jax.numpy as jnp
from jax.experimental import pallas as pl
from jax.experimental.pallas import tpu as pltpu


def _fused_body(x_ref, w_ref, adj_ref, out_ref, sup_ref, *, nk, tk):
    m = pl.program_id(1)
    k = pl.program_id(2)

    @pl.when((m == 0) & (k == 0))
    def _():
        sup_ref[...] = jnp.dot(
            x_ref[...], w_ref[...], preferred_element_type=jnp.float32
        ).astype(jnp.bfloat16)

    @pl.when(k == 0)
    def _():
        out_ref[...] = jnp.zeros_like(out_ref)

    adj_blk = adj_ref[...].astype(jnp.bfloat16)
    sup_blk = sup_ref[pl.ds(k * tk, tk), :]
    out_ref[...] += jnp.dot(adj_blk, sup_blk,
                            preferred_element_type=jnp.float32)

    @pl.when(k == nk - 1)
    def _():
        out_ref[...] = jnp.maximum(out_ref[...], 0.0)


@jax.jit
def kernel(input, adj, W):
    n_nodes, in_features = input.shape
    out_features = W.shape[1]

    tm, tk = 1024, 4096
    ncore = 2
    nm = n_nodes // tm // ncore
    nk = n_nodes // tk

    out = pl.pallas_call(
        functools.partial(_fused_body, nk=nk, tk=tk),
        grid=(ncore, nm, nk),
        in_specs=[
            pl.BlockSpec((n_nodes, in_features), lambda c, m, k: (0, 0)),
            pl.BlockSpec((in_features, out_features), lambda c, m, k: (0, 0)),
            pl.BlockSpec((tm, tk), lambda c, m, k, nm=nm: (c * nm + m, k)),
        ],
        out_specs=pl.BlockSpec(
            (tm, out_features), lambda c, m, k, nm=nm: (c * nm + m, 0)),
        out_shape=jax.ShapeDtypeStruct((n_nodes, out_features), jnp.float32),
        scratch_shapes=[pltpu.VMEM((n_nodes, out_features), jnp.bfloat16)],
        compiler_params=pltpu.CompilerParams(
            dimension_semantics=("parallel", "arbitrary", "arbitrary")),
    )(input, W, adj)
    return out


# speedup vs baseline: 1.2425x; 1.0041x over previous
import functools

import jax
import jax.numpy as jnp
from jax.experimental import pallas as pl
from jax.experimental.pallas import tpu as pltpu


def _fused_body(x_ref, w_ref, adj_ref, out_ref, sup_ref, *, nk, tk):
    m = pl.program_id(1)
    k = pl.program_id(2)

    @pl.when((m == 0) & (k == 0))
    def _():
        sup_ref[...] = jnp.dot(
            x_ref[...], w_ref[...], preferred_element_type=jnp.float32
        ).astype(jnp.bfloat16)

    @pl.when(k == 0)
    def _():
        out_ref[...] = jnp.zeros_like(out_ref)

    adj_blk = adj_ref[...].astype(jnp.bfloat16)
    sup_blk = sup_ref[pl.ds(k * tk, tk), :]
    out_ref[...] += jnp.dot(adj_blk, sup_blk,
                            preferred_element_type=jnp.float32)

    @pl.when(k == nk - 1)
    def _():
        out_ref[...] = jnp.maximum(out_ref[...], 0.0)


@jax.jit
def kernel(input, adj, W):
    n_nodes, in_features = input.shape
    out_features = W.shape[1]

    tm, tk = 512, 8192
    ncore = 2
    nm = n_nodes // tm // ncore
    nk = n_nodes // tk

    out = pl.pallas_call(
        functools.partial(_fused_body, nk=nk, tk=tk),
        grid=(ncore, nm, nk),
        in_specs=[
            pl.BlockSpec((n_nodes, in_features), lambda c, m, k: (0, 0)),
            pl.BlockSpec((in_features, out_features), lambda c, m, k: (0, 0)),
            pl.BlockSpec((tm, tk), lambda c, m, k, nm=nm: (c * nm + m, k)),
        ],
        out_specs=pl.BlockSpec(
            (tm, out_features), lambda c, m, k, nm=nm: (c * nm + m, 0)),
        out_shape=jax.ShapeDtypeStruct((n_nodes, out_features), jnp.float32),
        scratch_shapes=[pltpu.VMEM((n_nodes, out_features), jnp.bfloat16)],
        compiler_params=pltpu.CompilerParams(
            dimension_semantics=("parallel", "arbitrary", "arbitrary")),
    )(input, W, adj)
    return out


# speedup vs baseline: 1.2444x; 1.0015x over previous
import functools

import jax
import jax.numpy as jnp
from jax.experimental import pallas as pl
from jax.experimental.pallas import tpu as pltpu


def _fused_body(x_ref, w_ref, adj_ref, out_ref, sup_ref, *, nk, tk):
    m = pl.program_id(1)
    k = pl.program_id(2)

    @pl.when((m == 0) & (k == 0))
    def _():
        sup_ref[...] = jnp.dot(
            x_ref[...], w_ref[...], preferred_element_type=jnp.float32
        ).astype(jnp.bfloat16)

    @pl.when(k == 0)
    def _():
        out_ref[...] = jnp.zeros_like(out_ref)

    adj_blk = adj_ref[...].astype(jnp.bfloat16)
    sup_blk = sup_ref[pl.ds(k * tk, tk), :]
    out_ref[...] += jnp.dot(adj_blk, sup_blk,
                            preferred_element_type=jnp.float32)

    @pl.when(k == nk - 1)
    def _():
        out_ref[...] = jnp.maximum(out_ref[...], 0.0)


@jax.jit
def kernel(input, adj, W):
    n_nodes, in_features = input.shape
    out_features = W.shape[1]

    tm, tk = 256, 8192
    ncore = 2
    nm = n_nodes // tm // ncore
    nk = n_nodes // tk

    out = pl.pallas_call(
        functools.partial(_fused_body, nk=nk, tk=tk),
        grid=(ncore, nm, nk),
        in_specs=[
            pl.BlockSpec((n_nodes, in_features), lambda c, m, k: (0, 0)),
            pl.BlockSpec((in_features, out_features), lambda c, m, k: (0, 0)),
            pl.BlockSpec((tm, tk), lambda c, m, k, nm=nm: (c * nm + m, k)),
        ],
        out_specs=pl.BlockSpec(
            (tm, out_features), lambda c, m, k, nm=nm: (c * nm + m, 0)),
        out_shape=jax.ShapeDtypeStruct((n_nodes, out_features), jnp.float32),
        scratch_shapes=[pltpu.VMEM((n_nodes, out_features), jnp.bfloat16)],
        compiler_params=pltpu.CompilerParams(
            dimension_semantics=("parallel", "arbitrary", "arbitrary")),
    )(input, W, adj)
    return out
